# Initial kernel scaffold; baseline (speedup 1.0000x reference)
import jax
import jax.numpy as jnp
from jax import lax
from jax.experimental import pallas as pl

_N_TOK = 1152
_DIM = 64
_K = 1024
_COMMIT_W = 0.25
_ENT_W = 0.1
_INV_T = 100.0
_EPS = 1e-05


def _vq_body(x_ref, cb_ref, q_ref, vq_ref, commit_ref, cbl_ref, ent_ref,
             idx_ref):
    flat = x_ref[...]
    cb = cb_ref[...]

    xc = lax.dot_general(flat, cb, (((1,), (1,)), ((), ())),
                         preferred_element_type=jnp.float32,
                         precision=lax.Precision.HIGHEST)
    x2 = jnp.sum(flat * flat, axis=1, keepdims=True)
    ones = jnp.ones((1, _DIM), jnp.float32)
    c2 = lax.dot_general(ones, cb * cb, (((1,), (1,)), ((), ())),
                         preferred_element_type=jnp.float32,
                         precision=lax.Precision.HIGHEST)
    d2 = jnp.maximum(x2 - 2.0 * xc + c2, 0.0)
    d = jnp.sqrt(d2)

    dmin = jnp.min(d, axis=1, keepdims=True)
    iota = lax.broadcasted_iota(jnp.int32, (_N_TOK, _K), 1)
    idx = jnp.min(jnp.where(d == dmin, iota, _K), axis=1,
                  keepdims=True)
    idx_ref[...] = idx

    onehot = (iota == idx).astype(jnp.float32)
    q = lax.dot_general(onehot, cb, (((1,), (0,)), ((), ())),
                        preferred_element_type=jnp.float32,
                        precision=lax.Precision.HIGHEST)
    q_ref[...] = flat + (q - flat)

    diff = flat - q
    mse = jnp.sum(diff * diff) / (_N_TOK * _DIM)
    codebook_loss = 0.5 * mse
    commit_loss = codebook_loss * _COMMIT_W

    zc = (dmin - d) * _INV_T
    e = jnp.exp(zc)
    s = jnp.sum(e, axis=1, keepdims=True)
    p = e / s
    logp = zc - jnp.log(s)
    sample_entropy = -jnp.sum(p * logp) / _N_TOK
    avg_p = jnp.sum(p, axis=0, keepdims=True) / _N_TOK
    avg_entropy = -jnp.sum(avg_p * jnp.log(avg_p + _EPS))
    entropy_loss = (sample_entropy - avg_entropy) * _ENT_W

    vq_ref[0, 0] = codebook_loss + commit_loss + entropy_loss
    commit_ref[0, 0] = commit_loss
    cbl_ref[0, 0] = codebook_loss
    ent_ref[0, 0] = entropy_loss


def kernel(x, codebook):
    flat = x.reshape(_N_TOK, _DIM)
    scalar = jax.ShapeDtypeStruct((1, 1), jnp.float32)
    out_shape = (
        jax.ShapeDtypeStruct((_N_TOK, _DIM), jnp.float32),
        scalar, scalar, scalar, scalar,
        jax.ShapeDtypeStruct((_N_TOK, 1), jnp.int32),
    )
    q, vq, commit, cbl, ent, idx = pl.pallas_call(
        _vq_body,
        out_shape=out_shape,
    )(flat, codebook)
    return (q.reshape(x.shape),
            vq.reshape(()), commit.reshape(()), cbl.reshape(()),
            ent.reshape(()),
            idx.reshape(x.shape[:-1]))


# speedup vs baseline: 4.4256x; 4.4256x over previous
import jax
import jax.numpy as jnp
from jax import lax
from jax.experimental import pallas as pl

_N_TOK = 1152
_DIM = 64
_K = 1024
_COMMIT_W = 0.25
_ENT_W = 0.1
_INV_T = 100.0
_EPS = 1e-05


def _vq_body(x_ref, cb_ref, q_ref, vq_ref, commit_ref, cbl_ref, ent_ref,
             idx_ref):
    flat = x_ref[...]
    cb = cb_ref[...]

    xc = lax.dot_general(flat, cb, (((1,), (1,)), ((), ())),
                         preferred_element_type=jnp.float32,
                         precision=lax.Precision.HIGHEST)
    x2 = jnp.sum(flat * flat, axis=1, keepdims=True)
    ones = jnp.ones((1, _DIM), jnp.float32)
    c2 = lax.dot_general(ones, cb * cb, (((1,), (1,)), ((), ())),
                         preferred_element_type=jnp.float32,
                         precision=lax.Precision.HIGHEST)
    d2 = jnp.maximum(x2 - 2.0 * xc + c2, 0.0)
    d = jnp.sqrt(d2)

    dmin = jnp.min(d, axis=1, keepdims=True)
    iota = lax.broadcasted_iota(jnp.int32, (_N_TOK, _K), 1)
    idx = jnp.min(jnp.where(d == dmin, iota, _K), axis=1,
                  keepdims=True)
    idx_ref[...] = idx

    onehot = (iota == idx).astype(jnp.float32)
    q = lax.dot_general(onehot, cb, (((1,), (0,)), ((), ())),
                        preferred_element_type=jnp.float32,
                        precision=lax.Precision.HIGHEST)
    q_ref[...] = flat + (q - flat)

    diff = flat - q
    mse = jnp.sum(diff * diff) / (_N_TOK * _DIM)
    codebook_loss = 0.5 * mse
    commit_loss = codebook_loss * _COMMIT_W

    zc = (dmin - d) * _INV_T
    e = jnp.exp(zc)
    s = jnp.sum(e, axis=1, keepdims=True)
    p = e / s
    logp = zc - jnp.log(s)
    sample_entropy = -jnp.sum(p * logp) / _N_TOK
    avg_p = jnp.sum(p, axis=0, keepdims=True) / _N_TOK
    avg_entropy = -jnp.sum(avg_p * jnp.log(avg_p + _EPS))
    entropy_loss = (sample_entropy - avg_entropy) * _ENT_W

    vq_ref[...] = jnp.reshape(codebook_loss + commit_loss + entropy_loss,
                              (1, 1))
    commit_ref[...] = jnp.reshape(commit_loss, (1, 1))
    cbl_ref[...] = jnp.reshape(codebook_loss, (1, 1))
    ent_ref[...] = jnp.reshape(entropy_loss, (1, 1))


def kernel(x, codebook):
    flat = x.reshape(_N_TOK, _DIM)
    scalar = jax.ShapeDtypeStruct((1, 1), jnp.float32)
    out_shape = (
        jax.ShapeDtypeStruct((_N_TOK, _DIM), jnp.float32),
        scalar, scalar, scalar, scalar,
        jax.ShapeDtypeStruct((_N_TOK, 1), jnp.int32),
    )
    q, vq, commit, cbl, ent, idx = pl.pallas_call(
        _vq_body,
        out_shape=out_shape,
    )(flat, codebook)
    return (q.reshape(x.shape),
            vq.reshape(()), commit.reshape(()), cbl.reshape(()),
            ent.reshape(()),
            idx.reshape(x.shape[:-1]))
